# Initial kernel scaffold; baseline (speedup 1.0000x reference)
import jax
import jax.numpy as jnp
from jax.experimental import pallas as pl


def kernel(input):
    raise NotImplementedError("write your pallas kernel here")


# speedup vs baseline: 8.5121x; 8.5121x over previous
import jax
import jax.numpy as jnp
from jax.experimental import pallas as pl

_K = 64
_ROWS_PER_BLOCK = 16


def _topk_mask_body(x_ref, o_ref):
    x = x_ref[...]
    b = jax.lax.bitcast_convert_type(x, jnp.int32)
    key = b ^ (jax.lax.shift_right_arithmetic(b, 31) & jnp.int32(0x7FFFFFFF))

    min_i32 = jnp.int32(-2147483648)

    def step(i, ub):
        bit = jnp.int32(31) - i
        cand = ub | jax.lax.shift_left(jnp.int32(1), bit)
        c_signed = cand ^ min_i32
        cnt = jnp.sum((key >= c_signed).astype(jnp.int32), axis=1, keepdims=True)
        return jnp.where(cnt >= _K, cand, ub)

    ub0 = jnp.zeros((x.shape[0], 1), jnp.int32)
    ub = jax.lax.fori_loop(0, 32, step, ub0)
    thresh_key = ub ^ min_i32
    o_ref[...] = jnp.where(key >= thresh_key, x, jnp.float32(0.0))


@jax.jit
def kernel(input):
    m, n = input.shape
    grid = (m // _ROWS_PER_BLOCK,)
    return pl.pallas_call(
        _topk_mask_body,
        grid=grid,
        in_specs=[pl.BlockSpec((_ROWS_PER_BLOCK, n), lambda i: (i, 0))],
        out_specs=pl.BlockSpec((_ROWS_PER_BLOCK, n), lambda i: (i, 0)),
        out_shape=jax.ShapeDtypeStruct((m, n), jnp.float32),
    )(input)
